# baseline (speedup 1.0000x reference)
import jax
import jax.numpy as jnp
from jax import lax
from jax.experimental import pallas as pl
from jax.experimental.pallas import tpu as pltpu
from jax.experimental.pallas import tpu_sc as plsc

B, N, D = 16, 2048, 512
TOTAL = B * N
NC, NS = 2, 16
NW = NC * NS
ROWS_PER_W = TOTAL // NW
CHUNK = 32
NCHUNK = ROWS_PER_W // CHUNK
NBUF = 5
LAG = 3
LANES = 16


def _prep_kernel(dur_ref, cum_ref, mask_ref):
    dur = dur_ref[:]
    lane = lax.broadcasted_iota(jnp.int32, (B, N), 1)
    sub = lax.broadcasted_iota(jnp.int32, (B, 1), 0)
    c = dur
    k = 1
    while k < N:
        c = c + jnp.where(lane >= k, pltpu.roll(c, k, axis=1), 0)
        k *= 2
    totals = c[:, N - 1:N]
    p = totals
    k = 1
    while k < B:
        p = p + jnp.where(sub >= k, pltpu.roll(p, k, axis=0), 0)
        k *= 2
    cum_ref[:] = (p - totals) + (c - dur)
    mask_ref[:] = lane >= totals


def _gather_kernel(x_hbm, cum_hbm, out_hbm, *scratch):
    cum_v, idx_v = scratch[0], scratch[1]
    bufs = scratch[2:2 + NBUF]
    gsems = scratch[2 + NBUF:2 + 2 * NBUF]
    ssems = scratch[2 + 2 * NBUF:]
    wid = lax.axis_index("s") * NC + lax.axis_index("c")
    base = wid * ROWS_PER_W

    pltpu.sync_copy(cum_hbm, cum_v)
    lane = lax.iota(jnp.int32, LANES)

    def _search_one(g):
        res = jnp.zeros((LANES,), jnp.int32)
        for s in (2 ** e for e in reversed(range(16))):
            cand = res + s
            val = plsc.load_gather(cum_v, [jnp.minimum(cand - 1, TOTAL - 1)])
            ok = (val <= g) & (cand <= TOTAL)
            res = jnp.where(ok, cand, res)
        tok = res - 1
        return (tok & (N - 1)) + (g - (g & (N - 1)))

    def search_body(v, _):
        g0v = base + v * (2 * LANES) + lane
        g1v = g0v + LANES
        idx_v[pl.ds(v * (2 * LANES), LANES)] = _search_one(g0v)
        idx_v[pl.ds(v * (2 * LANES) + LANES, LANES)] = _search_one(g1v)
        return ()

    GROUPS = 4
    CH_PER_G = NCHUNK // GROUPS
    PAIRS_PER_G = (ROWS_PER_W // (2 * LANES)) // GROUPS

    g_h = {}
    s_h = {}
    for t in range(NCHUNK + LAG):
        if t < NCHUNK and t % CH_PER_G == 0:
            grp = t // CH_PER_G
            lax.fori_loop(grp * PAIRS_PER_G, (grp + 1) * PAIRS_PER_G,
                          search_body, ())
        if t < NCHUNK:
            bi = t % NBUF
            if t >= NBUF:
                s_h[t - NBUF].wait()
            g_h[t] = pltpu.async_copy(
                x_hbm.at[idx_v.at[pl.ds(t * CHUNK, CHUNK)]], bufs[bi],
                gsems[bi])
        tc = t - LAG
        if 0 <= tc < NCHUNK:
            bc = tc % NBUF
            g_h[tc].wait()
            off = pl.multiple_of(base + tc * CHUNK, CHUNK)
            s_h[tc] = pltpu.async_copy(bufs[bc],
                                       out_hbm.at[pl.ds(off, CHUNK)],
                                       ssems[bc])
    for t in range(NCHUNK - NBUF, NCHUNK):
        s_h[t].wait()


@jax.jit
def _expand(x, durations):
    x_flat = x.reshape(TOTAL, D)

    cum, mel_mask = pl.pallas_call(
        _prep_kernel,
        out_shape=[
            jax.ShapeDtypeStruct((B, N), jnp.int32),
            jax.ShapeDtypeStruct((B, N), jnp.bool_),
        ],
    )(durations)

    mesh = plsc.VectorSubcoreMesh(core_axis_name="c", subcore_axis_name="s")
    out_flat = pl.kernel(
        _gather_kernel,
        mesh=mesh,
        out_type=jax.ShapeDtypeStruct((TOTAL, D), jnp.float32),
        compiler_params=pltpu.CompilerParams(needs_layout_passes=False, use_tc_tiling_on_sc=False),
        scratch_types=(
            [pltpu.VMEM((TOTAL,), jnp.int32),
             pltpu.VMEM((ROWS_PER_W,), jnp.int32)]
            + [pltpu.VMEM((CHUNK, D), jnp.float32)] * NBUF
            + [pltpu.SemaphoreType.DMA] * (2 * NBUF)
        ),
    )(x_flat, cum.reshape(TOTAL))

    return out_flat.reshape(B, N, D), mel_mask


def kernel(x, durations):
    return _expand(x, durations)


# speedup vs baseline: 2.5047x; 2.5047x over previous
import jax
import jax.numpy as jnp
from jax import lax
from jax.experimental import pallas as pl
from jax.experimental.pallas import tpu as pltpu
from jax.experimental.pallas import tpu_sc as plsc

B, N, D = 16, 2048, 512
TOTAL = B * N
NC, NS = 2, 16
NW = NC * NS
ROWS_PER_W = TOTAL // NW
CHUNK = 32
NCHUNK = ROWS_PER_W // CHUNK
NBUF = 5
LAG = 3
LANES = 16


def _prep_kernel(dur_ref, cum_ref, mask_ref):
    dur = dur_ref[:]
    lane = lax.broadcasted_iota(jnp.int32, (B, N), 1)
    sub = lax.broadcasted_iota(jnp.int32, (B, 1), 0)
    c = dur
    k = 1
    while k < N:
        c = c + jnp.where(lane >= k, pltpu.roll(c, k, axis=1), 0)
        k *= 2
    totals = c[:, N - 1:N]
    p = totals
    k = 1
    while k < B:
        p = p + jnp.where(sub >= k, pltpu.roll(p, k, axis=0), 0)
        k *= 2
    cum_ref[:] = (p - totals) + (c - dur)
    mask_ref[:] = lane >= totals


def _gather_kernel(x_hbm, cum_hbm, out_hbm, *scratch):
    cum_v, idx_v = scratch[0], scratch[1]
    bufs = scratch[2:2 + NBUF]
    gsems = scratch[2 + NBUF:2 + 2 * NBUF]
    ssems = scratch[2 + 2 * NBUF:]
    wid = lax.axis_index("s") * NC + lax.axis_index("c")
    base = wid * ROWS_PER_W

    pltpu.sync_copy(cum_hbm, cum_v)
    lane = lax.iota(jnp.int32, LANES)

    def _search_one(g):
        res = jnp.zeros((LANES,), jnp.int32)
        for s in (2 ** e for e in reversed(range(16))):
            cand = res + s
            val = plsc.load_gather(cum_v, [jnp.minimum(cand - 1, TOTAL - 1)])
            ok = (val <= g) & (cand <= TOTAL)
            res = jnp.where(ok, cand, res)
        tok = res - 1
        return (tok & (N - 1)) + (g - (g & (N - 1)))

    def search_body(v, _):
        g0v = base + v * (2 * LANES) + lane
        g1v = g0v + LANES
        idx_v[pl.ds(v * (2 * LANES), LANES)] = _search_one(g0v)
        idx_v[pl.ds(v * (2 * LANES) + LANES, LANES)] = _search_one(g1v)
        return ()

    GROUPS = 4
    CH_PER_G = NCHUNK // GROUPS
    PAIRS_PER_G = (ROWS_PER_W // (2 * LANES)) // GROUPS

    g_h = {}
    s_h = {}
    for t in range(NCHUNK + LAG):
        if t < NCHUNK and t % CH_PER_G == 0:
            grp = t // CH_PER_G
            lax.fori_loop(grp * PAIRS_PER_G, (grp + 1) * PAIRS_PER_G,
                          search_body, ())
        if t < NCHUNK:
            bi = t % NBUF
            if t >= NBUF:
                s_h[t - NBUF].wait()
            h = CHUNK // 2
            g_h[t] = (
                pltpu.async_copy(
                    x_hbm.at[idx_v.at[pl.ds(t * CHUNK, h)]],
                    bufs[bi].at[pl.ds(0, h)], gsems[bi]),
                pltpu.async_copy(
                    x_hbm.at[idx_v.at[pl.ds(t * CHUNK + h, h)]],
                    bufs[bi].at[pl.ds(h, h)], gsems[bi]),
            )
        tc = t - LAG
        if 0 <= tc < NCHUNK:
            bc = tc % NBUF
            for _gh in g_h[tc]:
                _gh.wait()
            off = pl.multiple_of(base + tc * CHUNK, CHUNK)
            s_h[tc] = pltpu.async_copy(bufs[bc],
                                       out_hbm.at[pl.ds(off, CHUNK)],
                                       ssems[bc])
    for t in range(NCHUNK - NBUF, NCHUNK):
        s_h[t].wait()


@jax.jit
def _expand(x, durations):
    x_flat = x.reshape(TOTAL, D)

    cum, mel_mask = pl.pallas_call(
        _prep_kernel,
        out_shape=[
            jax.ShapeDtypeStruct((B, N), jnp.int32),
            jax.ShapeDtypeStruct((B, N), jnp.bool_),
        ],
    )(durations)

    mesh = plsc.VectorSubcoreMesh(core_axis_name="c", subcore_axis_name="s")
    out_flat = pl.kernel(
        _gather_kernel,
        mesh=mesh,
        out_type=jax.ShapeDtypeStruct((TOTAL, D), jnp.float32),
        compiler_params=pltpu.CompilerParams(needs_layout_passes=False),
        scratch_types=(
            [pltpu.VMEM((TOTAL,), jnp.int32),
             pltpu.VMEM((ROWS_PER_W,), jnp.int32)]
            + [pltpu.VMEM((CHUNK, D), jnp.float32)] * NBUF
            + [pltpu.SemaphoreType.DMA] * (2 * NBUF)
        ),
    )(x_flat, cum.reshape(TOTAL))

    return out_flat.reshape(B, N, D), mel_mask


def kernel(x, durations):
    return _expand(x, durations)


# speedup vs baseline: 2.5225x; 1.0071x over previous
import jax
import jax.numpy as jnp
from jax import lax
from jax.experimental import pallas as pl
from jax.experimental.pallas import tpu as pltpu
from jax.experimental.pallas import tpu_sc as plsc

B, N, D = 16, 2048, 512
TOTAL = B * N
NC, NS = 2, 16
NW = NC * NS
ROWS_PER_W = TOTAL // NW
CHUNK = 32
NCHUNK = ROWS_PER_W // CHUNK
NBUF = 5
LAG = 3
LANES = 16


def _prep_kernel(dur_ref, cum_ref, mask_ref):
    dur = dur_ref[:]
    lane = lax.broadcasted_iota(jnp.int32, (B, N), 1)
    sub = lax.broadcasted_iota(jnp.int32, (B, 1), 0)
    c = dur
    k = 1
    while k < N:
        c = c + jnp.where(lane >= k, pltpu.roll(c, k, axis=1), 0)
        k *= 2
    totals = c[:, N - 1:N]
    p = totals
    k = 1
    while k < B:
        p = p + jnp.where(sub >= k, pltpu.roll(p, k, axis=0), 0)
        k *= 2
    cum_ref[:] = (p - totals) + (c - dur)
    mask_ref[:] = lane >= totals


def _gather_kernel(x_hbm, cum_hbm, out_hbm, *scratch):
    cum_v, idx_v = scratch[0], scratch[1]
    bufs = scratch[2:2 + NBUF]
    gsems = scratch[2 + NBUF:2 + 2 * NBUF]
    ssems = scratch[2 + 2 * NBUF:]
    wid = lax.axis_index("s") * NC + lax.axis_index("c")
    base = wid * ROWS_PER_W

    pltpu.sync_copy(cum_hbm, cum_v)
    lane = lax.iota(jnp.int32, LANES)

    def _search_one(g):
        res = jnp.zeros((LANES,), jnp.int32)
        for s in (2 ** e for e in reversed(range(16))):
            cand = res + s
            val = plsc.load_gather(cum_v, [jnp.minimum(cand - 1, TOTAL - 1)])
            ok = (val <= g) & (cand <= TOTAL)
            res = jnp.where(ok, cand, res)
        tok = res - 1
        return (tok & (N - 1)) + (g - (g & (N - 1)))

    def search_body(v, _):
        g0v = base + v * (2 * LANES) + lane
        g1v = g0v + LANES
        idx_v[pl.ds(v * (2 * LANES), LANES)] = _search_one(g0v)
        idx_v[pl.ds(v * (2 * LANES) + LANES, LANES)] = _search_one(g1v)
        return ()

    GROUPS = 4
    CH_PER_G = NCHUNK // GROUPS
    PAIRS_PER_G = (ROWS_PER_W // (2 * LANES)) // GROUPS

    g_h = {}
    s_h = {}
    for t in range(NCHUNK + LAG):
        if t < NCHUNK and t % CH_PER_G == 0:
            grp = t // CH_PER_G
            lax.fori_loop(grp * PAIRS_PER_G, (grp + 1) * PAIRS_PER_G,
                          search_body, ())
        if t < NCHUNK:
            bi = t % NBUF
            if t >= NBUF:
                s_h[t - NBUF].wait()
            g_h[t] = pltpu.async_copy(
                x_hbm.at[idx_v.at[pl.ds(t * CHUNK, CHUNK)]], bufs[bi],
                gsems[bi])
        tc = t - LAG
        if 0 <= tc < NCHUNK:
            bc = tc % NBUF
            g_h[tc].wait()
            off = pl.multiple_of(base + tc * CHUNK, CHUNK)
            s_h[tc] = pltpu.async_copy(bufs[bc],
                                       out_hbm.at[pl.ds(off, CHUNK)],
                                       ssems[bc])
    for t in range(NCHUNK - NBUF, NCHUNK):
        s_h[t].wait()


@jax.jit
def _expand(x, durations):
    x_flat = x.reshape(TOTAL, D)

    cum, mel_mask = pl.pallas_call(
        _prep_kernel,
        out_shape=[
            jax.ShapeDtypeStruct((B, N), jnp.int32),
            jax.ShapeDtypeStruct((B, N), jnp.bool_),
        ],
    )(durations)

    mesh = plsc.VectorSubcoreMesh(core_axis_name="c", subcore_axis_name="s")
    out_flat = pl.kernel(
        _gather_kernel,
        mesh=mesh,
        out_type=jax.ShapeDtypeStruct((TOTAL, D), jnp.float32),
        compiler_params=pltpu.CompilerParams(needs_layout_passes=False),
        scratch_types=(
            [pltpu.VMEM((TOTAL,), jnp.int32),
             pltpu.VMEM((ROWS_PER_W,), jnp.int32)]
            + [pltpu.VMEM((CHUNK, D), jnp.float32)] * NBUF
            + [pltpu.SemaphoreType.DMA] * (2 * NBUF)
        ),
    )(x_flat, cum.reshape(TOTAL))

    return out_flat.reshape(B, N, D), mel_mask


def kernel(x, durations):
    return _expand(x, durations)


# speedup vs baseline: 2.5377x; 1.0060x over previous
import jax
import jax.numpy as jnp
from jax import lax
from jax.experimental import pallas as pl
from jax.experimental.pallas import tpu as pltpu
from jax.experimental.pallas import tpu_sc as plsc

B, N, D = 16, 2048, 512
TOTAL = B * N
NC, NS = 2, 16
NW = NC * NS
ROWS_PER_W = TOTAL // NW
CHUNK = 32
NCHUNK = ROWS_PER_W // CHUNK
NBUF = 5
LAG = 3
LANES = 16


def _prep_kernel(dur_ref, cum_ref, mask_ref):
    dur = dur_ref[:]
    lane = lax.broadcasted_iota(jnp.int32, (B, N), 1)
    sub = lax.broadcasted_iota(jnp.int32, (B, 1), 0)
    c = dur
    k = 1
    while k < N:
        c = c + jnp.where(lane >= k, pltpu.roll(c, k, axis=1), 0)
        k *= 2
    totals = c[:, N - 1:N]
    p = totals
    k = 1
    while k < B:
        p = p + jnp.where(sub >= k, pltpu.roll(p, k, axis=0), 0)
        k *= 2
    cum_ref[:] = (p - totals) + (c - dur)
    mask_ref[:] = lane >= totals


def _gather_kernel(x_hbm, cum_hbm, out_hbm, *scratch):
    cum_v, idx_v = scratch[0], scratch[1]
    bufs = scratch[2:2 + NBUF]
    gsems = scratch[2 + NBUF:2 + 2 * NBUF]
    ssems = scratch[2 + 2 * NBUF:]
    wid = lax.axis_index("s") * NC + lax.axis_index("c")
    base = wid * ROWS_PER_W

    pltpu.sync_copy(cum_hbm, cum_v)
    lane = lax.iota(jnp.int32, LANES)

    def _search_one(g):
        res = jnp.zeros((LANES,), jnp.int32)
        for s in (2 ** e for e in reversed(range(16))):
            cand = res + s
            val = plsc.load_gather(cum_v, [jnp.minimum(cand - 1, TOTAL - 1)])
            ok = (val <= g) & (cand <= TOTAL)
            res = jnp.where(ok, cand, res)
        tok = res - 1
        return (tok & (N - 1)) + (g - (g & (N - 1)))

    def search_body(v, _):
        g0v = base + v * (2 * LANES) + lane
        g1v = g0v + LANES
        idx_v[pl.ds(v * (2 * LANES), LANES)] = _search_one(g0v)
        idx_v[pl.ds(v * (2 * LANES) + LANES, LANES)] = _search_one(g1v)
        return ()

    GROUP_SIZES = (1, 1, 2, 4, 8, 8, 8)
    assert sum(GROUP_SIZES) == NCHUNK
    group_at = {}
    start = 0
    for size in GROUP_SIZES:
        group_at[start] = (start, start + size)
        start += size

    g_h = {}
    s_h = {}
    for t in range(NCHUNK + LAG):
        if t in group_at:
            lax.fori_loop(group_at[t][0], group_at[t][1], search_body, ())
        if t < NCHUNK:
            bi = t % NBUF
            if t >= NBUF:
                s_h[t - NBUF].wait()
            g_h[t] = pltpu.async_copy(
                x_hbm.at[idx_v.at[pl.ds(t * CHUNK, CHUNK)]], bufs[bi],
                gsems[bi])
        tc = t - LAG
        if 0 <= tc < NCHUNK:
            bc = tc % NBUF
            g_h[tc].wait()
            off = pl.multiple_of(base + tc * CHUNK, CHUNK)
            s_h[tc] = pltpu.async_copy(bufs[bc],
                                       out_hbm.at[pl.ds(off, CHUNK)],
                                       ssems[bc])
    for t in range(NCHUNK - NBUF, NCHUNK):
        s_h[t].wait()


@jax.jit
def _expand(x, durations):
    x_flat = x.reshape(TOTAL, D)

    cum, mel_mask = pl.pallas_call(
        _prep_kernel,
        out_shape=[
            jax.ShapeDtypeStruct((B, N), jnp.int32),
            jax.ShapeDtypeStruct((B, N), jnp.bool_),
        ],
    )(durations)

    mesh = plsc.VectorSubcoreMesh(core_axis_name="c", subcore_axis_name="s")
    out_flat = pl.kernel(
        _gather_kernel,
        mesh=mesh,
        out_type=jax.ShapeDtypeStruct((TOTAL, D), jnp.float32),
        compiler_params=pltpu.CompilerParams(needs_layout_passes=False),
        scratch_types=(
            [pltpu.VMEM((TOTAL,), jnp.int32),
             pltpu.VMEM((ROWS_PER_W,), jnp.int32)]
            + [pltpu.VMEM((CHUNK, D), jnp.float32)] * NBUF
            + [pltpu.SemaphoreType.DMA] * (2 * NBUF)
        ),
    )(x_flat, cum.reshape(TOTAL))

    return out_flat.reshape(B, N, D), mel_mask


def kernel(x, durations):
    return _expand(x, durations)
